# baseline (speedup 1.0000x reference)
import jax
import jax.numpy as jnp
from jax import lax
from jax.experimental import pallas as pl
from jax.experimental.pallas import tpu as pltpu
from jax.experimental.pallas import tpu_sc as plsc

HID = 32
BATCH = 16384
TBL = 1000000
NTC = 7813
SLAB = NTC * 1024
FLAT = 4 * SLAB
NC = 2
NS = 16
NW = NC * NS
BPW = BATCH // NW

CHUNK = 1024
FULL_CHUNKS = TBL // CHUNK
TAIL = FULL_CHUNKS * CHUNK


def _detile_body(wt_hbm, flat_hbm, buf0, buf1, sem0, sem1, esem0, esem1):
    wid = lax.axis_index("s") * NC + lax.axis_index("c")
    n_own = (FULL_CHUNKS - wid + NW - 1) // NW

    bufs = (buf0, buf1)
    sems = (sem0, sem1)
    esems = (esem0, esem1)

    def start(k, b):
        c0 = pl.multiple_of((wid + k * NW) * CHUNK, 128)
        pltpu.async_copy(wt_hbm.at[:, pl.ds(c0, CHUNK)], bufs[b], sems[b])

    def wait_stream(b):
        pltpu.make_async_copy(
            wt_hbm.at[:, pl.ds(0, CHUNK)], bufs[b], sems[b]
        ).wait()

    def emit(k, b):
        c0 = (wid + k * NW) * CHUNK
        for cg in range(4):
            for lb in range(CHUNK // 128):
                t = cg * NTC + c0 // 128 + lb
                pltpu.async_copy(
                    bufs[b].at[pl.ds(8 * cg, 8), pl.ds(128 * lb, 128)],
                    flat_hbm.at[t],
                    esems[b],
                )

    def drain_emits(b):
        pltpu.make_async_copy(
            wt_hbm.at[:, pl.ds(0, CHUNK)], bufs[b], esems[b]
        ).wait()

    start(0, 0)

    def step(k, carry):
        b = k % 2

        @pl.when(b == 0)
        def _():
            @pl.when(k + 1 < n_own)
            def _():
                @pl.when(k >= 1)
                def _():
                    drain_emits(1)
                start(k + 1, 1)

            wait_stream(0)
            emit(k, 0)

        @pl.when(b == 1)
        def _():
            @pl.when(k + 1 < n_own)
            def _():
                drain_emits(0)
                start(k + 1, 0)

            wait_stream(1)
            emit(k, 1)

        return carry

    lax.fori_loop(0, n_own, step, 0)
    last = (n_own - 1) % 2

    @pl.when(last == 0)
    def _():
        drain_emits(0)
        drain_emits(1)

    @pl.when(last == 1)
    def _():
        drain_emits(1)
        drain_emits(0)


def _gather_body(idx_hbm, flat_hbm, tail_hbm, out_hbm,
                 idx_v, off_v, val_v, tail_v, sem):
    wid = lax.axis_index("s") * NC + lax.axis_index("c")
    base = wid * BPW
    pltpu.sync_copy(idx_hbm.at[pl.ds(base, BPW)], idx_v)
    pltpu.sync_copy(tail_hbm, tail_v)
    lane = lax.iota(jnp.int32, 16)

    def offsets(jv, carry):
        r = idx_v[pl.ds(pl.multiple_of(jv * 16, 16), 16)]
        cb = (r >> 7) * 1024 + (r & 127)
        for c in range(HID):
            o = cb + ((c >> 3) * SLAB + (c & 7) * 128)
            pos = jv * 512 + lane * 32 + c
            plsc.store_scatter(off_v, [pos], o)
        return carry

    lax.fori_loop(0, BPW // 16, offsets, 0)
    pltpu.async_copy(flat_hbm.at[off_v], val_v, sem).wait()

    def fixup(jv, carry):
        r = idx_v[pl.ds(pl.multiple_of(jv * 16, 16), 16)]
        m = r >= TAIL
        any_hit = jnp.max(jnp.where(m, jnp.int32(1), jnp.int32(0)))

        @pl.when(any_hit > 0)
        def _():
            rm = jnp.clip(r - TAIL, 0, TBL - TAIL - 1)
            for c in range(HID):
                tv = plsc.load_gather(
                    tail_v, [rm, jnp.full((16,), c, jnp.int32)]
                )
                pos = jv * 512 + lane * 32 + c
                plsc.store_scatter(val_v, [pos], tv, mask=m)

        return carry

    lax.fori_loop(0, BPW // 16, fixup, 0)
    pltpu.sync_copy(
        val_v, out_hbm.at[pl.ds(base * HID, BPW * HID)]
    )


def kernel(ndata_id, w2v_embed):
    mesh = plsc.VectorSubcoreMesh(core_axis_name="c", subcore_axis_name="s")
    detile = pl.kernel(
        _detile_body,
        out_type=jax.ShapeDtypeStruct((4 * NTC, 8, 128), jnp.float32),
        mesh=mesh,
        scratch_types=[
            pltpu.VMEM((HID, CHUNK), jnp.float32),
            pltpu.VMEM((HID, CHUNK), jnp.float32),
            pltpu.SemaphoreType.DMA,
            pltpu.SemaphoreType.DMA,
            pltpu.SemaphoreType.DMA,
            pltpu.SemaphoreType.DMA,
        ],
        compiler_params=pltpu.CompilerParams(use_tc_tiling_on_sc=True, needs_layout_passes=False),
    )
    gather = pl.kernel(
        _gather_body,
        out_type=jax.ShapeDtypeStruct((BATCH * HID,), jnp.float32),
        mesh=mesh,
        scratch_types=[
            pltpu.VMEM((BPW,), jnp.int32),
            pltpu.VMEM((BPW * HID,), jnp.int32),
            pltpu.VMEM((BPW * HID,), jnp.float32),
            pltpu.VMEM((TBL - TAIL, HID), jnp.float32),
            pltpu.SemaphoreType.DMA,
        ],
        compiler_params=pltpu.CompilerParams(use_tc_tiling_on_sc=False, needs_layout_passes=False),
    )
    idx = ndata_id.astype(jnp.int32)
    flat = detile(w2v_embed.T)
    tail = w2v_embed[TAIL:]
    out_flat = gather(idx, flat.reshape(FLAT), tail)
    return out_flat.reshape(BATCH, HID)


# speedup vs baseline: 1.0121x; 1.0121x over previous
import jax
import jax.numpy as jnp
from jax import lax
from jax.experimental import pallas as pl
from jax.experimental.pallas import tpu as pltpu
from jax.experimental.pallas import tpu_sc as plsc

HID = 32
BATCH = 16384
TBL = 1000000
NTC = 7813
SLAB = NTC * 1024
FLAT = 4 * SLAB
NC = 2
NS = 16
NW = NC * NS
BPW = BATCH // NW

CHUNK = 512
FULL_CHUNKS = TBL // CHUNK
TAIL = FULL_CHUNKS * CHUNK


def _detile_body(wt_hbm, flat_hbm, buf0, buf1, sem0, sem1, esem0, esem1):
    wid = lax.axis_index("s") * NC + lax.axis_index("c")
    n_own = (FULL_CHUNKS - wid + NW - 1) // NW

    bufs = (buf0, buf1)
    sems = (sem0, sem1)
    esems = (esem0, esem1)

    def start(k, b):
        c0 = pl.multiple_of((wid + k * NW) * CHUNK, 128)
        pltpu.async_copy(wt_hbm.at[:, pl.ds(c0, CHUNK)], bufs[b], sems[b])

    def wait_stream(b):
        pltpu.make_async_copy(
            wt_hbm.at[:, pl.ds(0, CHUNK)], bufs[b], sems[b]
        ).wait()

    def emit(k, b):
        c0 = (wid + k * NW) * CHUNK
        for cg in range(4):
            for lb in range(CHUNK // 128):
                t = cg * NTC + c0 // 128 + lb
                pltpu.async_copy(
                    bufs[b].at[pl.ds(8 * cg, 8), pl.ds(128 * lb, 128)],
                    flat_hbm.at[t],
                    esems[b],
                )

    def drain_emits(b):
        pltpu.make_async_copy(
            wt_hbm.at[:, pl.ds(0, CHUNK)], bufs[b], esems[b]
        ).wait()

    start(0, 0)

    def step(k, carry):
        b = k % 2

        @pl.when(b == 0)
        def _():
            @pl.when(k + 1 < n_own)
            def _():
                @pl.when(k >= 1)
                def _():
                    drain_emits(1)
                start(k + 1, 1)

            wait_stream(0)
            emit(k, 0)

        @pl.when(b == 1)
        def _():
            @pl.when(k + 1 < n_own)
            def _():
                drain_emits(0)
                start(k + 1, 0)

            wait_stream(1)
            emit(k, 1)

        return carry

    lax.fori_loop(0, n_own, step, 0)
    last = (n_own - 1) % 2

    @pl.when(last == 0)
    def _():
        drain_emits(0)
        drain_emits(1)

    @pl.when(last == 1)
    def _():
        drain_emits(1)
        drain_emits(0)


def _gather_body(idx_hbm, flat_hbm, tail_hbm, out_hbm,
                 idx_v, off_v, val_v, tail_v, sem):
    wid = lax.axis_index("s") * NC + lax.axis_index("c")
    base = wid * BPW
    pltpu.sync_copy(idx_hbm.at[pl.ds(base, BPW)], idx_v)
    pltpu.sync_copy(tail_hbm, tail_v)
    lane = lax.iota(jnp.int32, 16)

    def offsets(jv, carry):
        r = idx_v[pl.ds(pl.multiple_of(jv * 16, 16), 16)]
        cb = (r >> 7) * 1024 + (r & 127)
        for c in range(HID):
            o = cb + ((c >> 3) * SLAB + (c & 7) * 128)
            pos = jv * 512 + lane * 32 + c
            plsc.store_scatter(off_v, [pos], o)
        return carry

    lax.fori_loop(0, BPW // 16, offsets, 0)
    pltpu.async_copy(flat_hbm.at[off_v], val_v, sem).wait()

    def fixup(jv, carry):
        r = idx_v[pl.ds(pl.multiple_of(jv * 16, 16), 16)]
        m = r >= TAIL
        any_hit = jnp.max(jnp.where(m, jnp.int32(1), jnp.int32(0)))

        @pl.when(any_hit > 0)
        def _():
            rm = jnp.clip(r - TAIL, 0, TBL - TAIL - 1)
            for c in range(HID):
                tv = plsc.load_gather(
                    tail_v, [rm, jnp.full((16,), c, jnp.int32)]
                )
                pos = jv * 512 + lane * 32 + c
                plsc.store_scatter(val_v, [pos], tv, mask=m)

        return carry

    lax.fori_loop(0, BPW // 16, fixup, 0)
    pltpu.sync_copy(
        val_v, out_hbm.at[pl.ds(base * HID, BPW * HID)]
    )


def kernel(ndata_id, w2v_embed):
    mesh = plsc.VectorSubcoreMesh(core_axis_name="c", subcore_axis_name="s")
    detile = pl.kernel(
        _detile_body,
        out_type=jax.ShapeDtypeStruct((4 * NTC, 8, 128), jnp.float32),
        mesh=mesh,
        scratch_types=[
            pltpu.VMEM((HID, CHUNK), jnp.float32),
            pltpu.VMEM((HID, CHUNK), jnp.float32),
            pltpu.SemaphoreType.DMA,
            pltpu.SemaphoreType.DMA,
            pltpu.SemaphoreType.DMA,
            pltpu.SemaphoreType.DMA,
        ],
        compiler_params=pltpu.CompilerParams(use_tc_tiling_on_sc=True, needs_layout_passes=False),
    )
    gather = pl.kernel(
        _gather_body,
        out_type=jax.ShapeDtypeStruct((BATCH * HID,), jnp.float32),
        mesh=mesh,
        scratch_types=[
            pltpu.VMEM((BPW,), jnp.int32),
            pltpu.VMEM((BPW * HID,), jnp.int32),
            pltpu.VMEM((BPW * HID,), jnp.float32),
            pltpu.VMEM((TBL - TAIL, HID), jnp.float32),
            pltpu.SemaphoreType.DMA,
        ],
        compiler_params=pltpu.CompilerParams(use_tc_tiling_on_sc=False, needs_layout_passes=False),
    )
    idx = ndata_id.astype(jnp.int32)
    flat = detile(w2v_embed.T)
    tail = w2v_embed[TAIL:]
    out_flat = gather(idx, flat.reshape(FLAT), tail)
    return out_flat.reshape(BATCH, HID)
